# Initial kernel scaffold; baseline (speedup 1.0000x reference)
import jax
import jax.numpy as jnp
from jax.experimental import pallas as pl


def kernel(predict_pc, gt_pc):
    raise NotImplementedError("write your pallas kernel here")


# speedup vs baseline: 1.7785x; 1.7785x over previous
import functools

import jax
import jax.numpy as jnp
from jax.experimental import pallas as pl

_EPS = 1e-8


def _chamfer_kernel(pT_ref, g_ref, out_ref, *, tm: int, m: int, n: int):
    gx = g_ref[0, 0:1, :]
    gy = g_ref[0, 1:2, :]
    gz = g_ref[0, 2:3, :]
    g_sq = gx * gx + gy * gy + gz * gz
    gb = g_ref[0].astype(jnp.bfloat16)

    num_tiles = m // tm

    def body(i, carry):
        fsum, colmin_a, colsel_e = carry
        px = pT_ref[0, pl.ds(i * tm, tm), 0:1]
        py = pT_ref[0, pl.ds(i * tm, tm), 1:2]
        pz = pT_ref[0, pl.ds(i * tm, tm), 2:3]
        p_sq = px * px + py * py + pz * pz
        pb = pT_ref[0, pl.ds(i * tm, tm), :].astype(jnp.bfloat16)
        cross = jax.lax.dot_general(
            pb, gb, dimension_numbers=(((1,), (0,)), ((), ())),
            preferred_element_type=jnp.float32)
        d2a = p_sq - 2.0 * cross + g_sq
        dx = px - gx
        d2e = dx * dx
        dy = py - gy
        d2e = d2e + dy * dy
        dz = pz - gz
        d2e = d2e + dz * dz

        rowmin_a = jnp.min(d2a, axis=1, keepdims=True)
        rowsel_e = jnp.min(jnp.where(d2a == rowmin_a, d2e, jnp.inf), axis=1)
        fsum = fsum + jnp.sum(jnp.sqrt(rowsel_e + _EPS))

        tile_cmin_a = jnp.min(d2a, axis=0, keepdims=True)
        tile_csel_e = jnp.min(
            jnp.where(d2a == tile_cmin_a, d2e, jnp.inf), axis=0, keepdims=True)
        take_new = tile_cmin_a < colmin_a
        colsel_e = jnp.where(take_new, tile_csel_e, colsel_e)
        colmin_a = jnp.where(take_new, tile_cmin_a, colmin_a)
        return fsum, colmin_a, colsel_e

    init = (jnp.float32(0.0),
            jnp.full((1, n), jnp.inf, dtype=jnp.float32),
            jnp.full((1, n), jnp.inf, dtype=jnp.float32))
    fsum, _, colsel_e = jax.lax.fori_loop(0, num_tiles, body, init)
    bsum = jnp.sum(jnp.sqrt(colsel_e + _EPS))
    row = jax.lax.broadcasted_iota(jnp.int32, (8, 128), 0)
    col = jax.lax.broadcasted_iota(jnp.int32, (8, 128), 1)
    out = jnp.where((row == 0) & (col == 0), fsum,
                    jnp.where((row == 0) & (col == 1), bsum, 0.0))
    out_ref[0] = out


@jax.jit
def kernel(predict_pc, gt_pc):
    b, _, m = predict_pc.shape
    n = gt_pc.shape[2]
    tm = 256
    pT = jnp.transpose(predict_pc[:, :3, :], (0, 2, 1))
    g = gt_pc[:, :3, :]
    out = pl.pallas_call(
        functools.partial(_chamfer_kernel, tm=tm, m=m, n=n),
        grid=(b,),
        in_specs=[
            pl.BlockSpec((1, m, 3), lambda i: (i, 0, 0)),
            pl.BlockSpec((1, 3, n), lambda i: (i, 0, 0)),
        ],
        out_specs=pl.BlockSpec((1, 8, 128), lambda i: (i, 0, 0)),
        out_shape=jax.ShapeDtypeStruct((b, 8, 128), jnp.float32),
    )(pT, g)
    forward = jnp.sum(out[:, 0, 0]) / (b * m)
    backward = jnp.sum(out[:, 0, 1]) / (b * n)
    return forward + backward


# speedup vs baseline: 1.9114x; 1.0747x over previous
import functools

import jax
import jax.numpy as jnp
from jax.experimental import pallas as pl

_EPS = 1e-8


def _chamfer_kernel(pT_ref, g_ref, out_ref, *, tm: int, m: int, n: int):
    gx = g_ref[0, 0:1, :]
    gy = g_ref[0, 1:2, :]
    gz = g_ref[0, 2:3, :]
    g_sq = gx * gx + gy * gy + gz * gz
    gb = g_ref[0].astype(jnp.bfloat16)

    num_tiles = m // tm

    def body(i, carry):
        fsum, colmin_a, colsel_e = carry
        px = pT_ref[0, pl.ds(i * tm, tm), 0:1]
        py = pT_ref[0, pl.ds(i * tm, tm), 1:2]
        pz = pT_ref[0, pl.ds(i * tm, tm), 2:3]
        p_sq = px * px + py * py + pz * pz
        pb = pT_ref[0, pl.ds(i * tm, tm), :].astype(jnp.bfloat16)
        cross = jax.lax.dot_general(
            pb, gb, dimension_numbers=(((1,), (0,)), ((), ())),
            preferred_element_type=jnp.float32)
        d2a = p_sq - 2.0 * cross + g_sq
        dx = px - gx
        d2e = dx * dx
        dy = py - gy
        d2e = d2e + dy * dy
        dz = pz - gz
        d2e = d2e + dz * dz

        rowmin_a = jnp.min(d2a, axis=1, keepdims=True)
        rowsel_e = jnp.min(jnp.where(d2a == rowmin_a, d2e, jnp.inf), axis=1)
        fsum = fsum + jnp.sum(jnp.sqrt(rowsel_e + _EPS))

        tile_cmin_a = jnp.min(d2a, axis=0, keepdims=True)
        tile_csel_e = jnp.min(
            jnp.where(d2a == tile_cmin_a, d2e, jnp.inf), axis=0, keepdims=True)
        take_new = tile_cmin_a < colmin_a
        colsel_e = jnp.where(take_new, tile_csel_e, colsel_e)
        colmin_a = jnp.where(take_new, tile_cmin_a, colmin_a)
        return fsum, colmin_a, colsel_e

    init = (jnp.float32(0.0),
            jnp.full((1, n), jnp.inf, dtype=jnp.float32),
            jnp.full((1, n), jnp.inf, dtype=jnp.float32))
    fsum, _, colsel_e = jax.lax.fori_loop(0, num_tiles, body, init)
    bsum = jnp.sum(jnp.sqrt(colsel_e + _EPS))
    row = jax.lax.broadcasted_iota(jnp.int32, (8, 128), 0)
    col = jax.lax.broadcasted_iota(jnp.int32, (8, 128), 1)
    out = jnp.where((row == 0) & (col == 0), fsum,
                    jnp.where((row == 0) & (col == 1), bsum, 0.0))
    out_ref[0] = out


@jax.jit
def kernel(predict_pc, gt_pc):
    b, _, m = predict_pc.shape
    n = gt_pc.shape[2]
    tm = 512
    pT = jnp.transpose(predict_pc[:, :3, :], (0, 2, 1))
    g = gt_pc[:, :3, :]
    out = pl.pallas_call(
        functools.partial(_chamfer_kernel, tm=tm, m=m, n=n),
        grid=(b,),
        in_specs=[
            pl.BlockSpec((1, m, 3), lambda i: (i, 0, 0)),
            pl.BlockSpec((1, 3, n), lambda i: (i, 0, 0)),
        ],
        out_specs=pl.BlockSpec((1, 8, 128), lambda i: (i, 0, 0)),
        out_shape=jax.ShapeDtypeStruct((b, 8, 128), jnp.float32),
    )(pT, g)
    forward = jnp.sum(out[:, 0, 0]) / (b * m)
    backward = jnp.sum(out[:, 0, 1]) / (b * n)
    return forward + backward
